# baseline (speedup 1.0000x reference)
import functools

import jax
import jax.numpy as jnp
from jax import lax
from jax.experimental import pallas as pl
from jax.experimental.pallas import tpu as pltpu
from jax.experimental.pallas import tpu_sc as plsc

VOCAB = 100000
EMBED_DIM = 64
BATCH = 1024
CTX = 20

NUM_CORES = 2
NUM_SUBCORES = 16
NW = NUM_CORES * NUM_SUBCORES
B_PER_W = BATCH // NW
IDX_PER_W = B_PER_W * CTX
IDX_CHUNK = 128
N_CHUNKS = IDX_PER_W // IDX_CHUNK
D_SLICES = EMBED_DIM // 16

VT = 2048


def _gather_sum_body(idx_hbm, table_hbm, out_hbm, idx_v, rows_v, acc_v, sem):
    wid = lax.axis_index("s") * NUM_CORES + lax.axis_index("c")

    pltpu.sync_copy(idx_hbm.at[wid], idx_v)

    copies = []
    for j in range(N_CHUNKS):
        copies.append(
            pltpu.async_copy(
                table_hbm.at[idx_v.at[j]],
                rows_v.at[pl.ds(j * IDX_CHUNK, IDX_CHUNK)],
                sem,
            )
        )
    for c in copies:
        c.wait()

    def reduce_one(b, _):
        base = b * CTX
        for d in range(D_SLICES):
            sl = pl.ds(d * 16, 16)
            acc = rows_v[base, sl]
            for c in range(1, CTX):
                acc = acc + rows_v[base + c, sl]
            acc_v[b, sl] = acc
        return 0

    lax.fori_loop(0, B_PER_W, reduce_one, 0)

    pltpu.sync_copy(acc_v, out_hbm.at[pl.ds(wid * B_PER_W, B_PER_W)])


@functools.cache
def _gather_sum():
    return pl.kernel(
        _gather_sum_body,
        out_type=jax.ShapeDtypeStruct((BATCH, EMBED_DIM), jnp.float32),
        mesh=plsc.VectorSubcoreMesh(core_axis_name="c", subcore_axis_name="s"),
        compiler_params=pltpu.CompilerParams(use_tc_tiling_on_sc=False),
        scratch_types=[
            pltpu.VMEM((N_CHUNKS, IDX_CHUNK), jnp.int32),
            pltpu.VMEM((IDX_PER_W, EMBED_DIM), jnp.float32),
            pltpu.VMEM((B_PER_W, EMBED_DIM), jnp.float32),
            pltpu.SemaphoreType.DMA,
        ],
    )


def _matmul_body(e_ref, w_ref, b_ref, o_ref):
    o_ref[...] = (
        lax.dot_general(
            e_ref[...],
            w_ref[...],
            dimension_numbers=(((1,), (1,)), ((), ())),
            preferred_element_type=jnp.float32,
        )
        + b_ref[...]
    )


def _project(embeds, linear_w, linear_b2d):
    n_vt = pl.cdiv(VOCAB, VT)
    return pl.pallas_call(
        _matmul_body,
        grid=(n_vt,),
        in_specs=[
            pl.BlockSpec((BATCH, EMBED_DIM), lambda j: (0, 0)),
            pl.BlockSpec((VT, EMBED_DIM), lambda j: (j, 0)),
            pl.BlockSpec((1, VT), lambda j: (0, j)),
        ],
        out_specs=pl.BlockSpec((BATCH, VT), lambda j: (0, j)),
        out_shape=jax.ShapeDtypeStruct((BATCH, VOCAB), jnp.float32),
    )(embeds, linear_w, linear_b2d)


def kernel(inputs, labels, embed_table, linear_w, linear_b):
    del labels
    idx = inputs.astype(jnp.int32).reshape(NW, N_CHUNKS, IDX_CHUNK)
    embeds = _gather_sum()(idx, embed_table)
    return _project(embeds, linear_w, linear_b.reshape(1, VOCAB))


# speedup vs baseline: 1.0054x; 1.0054x over previous
import functools

import jax
import jax.numpy as jnp
from jax import lax
from jax.experimental import pallas as pl
from jax.experimental.pallas import tpu as pltpu
from jax.experimental.pallas import tpu_sc as plsc

VOCAB = 100000
EMBED_DIM = 64
BATCH = 1024
CTX = 20

NUM_CORES = 2
NUM_SUBCORES = 16
NW = NUM_CORES * NUM_SUBCORES
B_PER_W = BATCH // NW
IDX_PER_W = B_PER_W * CTX
IDX_CHUNK = 128
N_CHUNKS = IDX_PER_W // IDX_CHUNK
D_SLICES = EMBED_DIM // 16

VT = 4096


def _gather_sum_body(idx_hbm, table_hbm, out_hbm, idx_v, rows_v, acc_v, sem):
    wid = lax.axis_index("s") * NUM_CORES + lax.axis_index("c")

    pltpu.sync_copy(idx_hbm.at[wid], idx_v)

    copies = []
    for j in range(N_CHUNKS):
        copies.append(
            pltpu.async_copy(
                table_hbm.at[idx_v.at[j]],
                rows_v.at[pl.ds(j * IDX_CHUNK, IDX_CHUNK)],
                sem,
            )
        )
    for c in copies:
        c.wait()

    def reduce_one(b, _):
        base = b * CTX
        for d in range(D_SLICES):
            sl = pl.ds(d * 16, 16)
            acc = rows_v[base, sl]
            for c in range(1, CTX):
                acc = acc + rows_v[base + c, sl]
            acc_v[b, sl] = acc
        return 0

    lax.fori_loop(0, B_PER_W, reduce_one, 0)

    pltpu.sync_copy(acc_v, out_hbm.at[pl.ds(wid * B_PER_W, B_PER_W)])


@functools.cache
def _gather_sum():
    return pl.kernel(
        _gather_sum_body,
        out_type=jax.ShapeDtypeStruct((BATCH, EMBED_DIM), jnp.float32),
        mesh=plsc.VectorSubcoreMesh(core_axis_name="c", subcore_axis_name="s"),
        compiler_params=pltpu.CompilerParams(use_tc_tiling_on_sc=False),
        scratch_types=[
            pltpu.VMEM((N_CHUNKS, IDX_CHUNK), jnp.int32),
            pltpu.VMEM((IDX_PER_W, EMBED_DIM), jnp.float32),
            pltpu.VMEM((B_PER_W, EMBED_DIM), jnp.float32),
            pltpu.SemaphoreType.DMA,
        ],
    )


def _matmul_body(e_ref, w_ref, b_ref, o_ref):
    o_ref[...] = (
        lax.dot_general(
            e_ref[...],
            w_ref[...],
            dimension_numbers=(((1,), (1,)), ((), ())),
            preferred_element_type=jnp.float32,
        )
        + b_ref[...]
    )


def _project(embeds, linear_w, linear_b2d):
    n_vt = pl.cdiv(VOCAB, VT)
    return pl.pallas_call(
        _matmul_body,
        grid=(n_vt,),
        in_specs=[
            pl.BlockSpec((BATCH, EMBED_DIM), lambda j: (0, 0)),
            pl.BlockSpec((VT, EMBED_DIM), lambda j: (j, 0)),
            pl.BlockSpec((1, VT), lambda j: (0, j)),
        ],
        out_specs=pl.BlockSpec((BATCH, VT), lambda j: (0, j)),
        out_shape=jax.ShapeDtypeStruct((BATCH, VOCAB), jnp.float32),
    )(embeds, linear_w, linear_b2d)


def kernel(inputs, labels, embed_table, linear_w, linear_b):
    del labels
    idx = inputs.astype(jnp.int32).reshape(NW, N_CHUNKS, IDX_CHUNK)
    embeds = _gather_sum()(idx, embed_table)
    return _project(embeds, linear_w, linear_b.reshape(1, VOCAB))


# speedup vs baseline: 1.0355x; 1.0299x over previous
import functools

import jax
import jax.numpy as jnp
from jax import lax
from jax.experimental import pallas as pl
from jax.experimental.pallas import tpu as pltpu
from jax.experimental.pallas import tpu_sc as plsc

VOCAB = 100000
EMBED_DIM = 64
BATCH = 1024
CTX = 20

NUM_CORES = 2
NUM_SUBCORES = 16
NW = NUM_CORES * NUM_SUBCORES
B_PER_W = BATCH // NW
IDX_PER_W = B_PER_W * CTX
IDX_CHUNK = 128
N_CHUNKS = IDX_PER_W // IDX_CHUNK
D_SLICES = EMBED_DIM // 16

VT = 4096


def _gather_sum_body(idx_hbm, table_hbm, out_hbm, idx_v, rows_v, acc_v, sem):
    wid = lax.axis_index("s") * NUM_CORES + lax.axis_index("c")

    pltpu.sync_copy(idx_hbm.at[wid], idx_v)

    copies = []
    for j in range(N_CHUNKS):
        copies.append(
            pltpu.async_copy(
                table_hbm.at[idx_v.at[j]],
                rows_v.at[pl.ds(j * IDX_CHUNK, IDX_CHUNK)],
                sem,
            )
        )
    for c in copies:
        c.wait()

    def reduce_one(b, _):
        base = b * CTX
        for d in range(D_SLICES):
            sl = pl.ds(d * 16, 16)
            acc = rows_v[base, sl]
            for c in range(1, CTX):
                acc = acc + rows_v[base + c, sl]
            acc_v[b, sl] = acc
        return 0

    lax.fori_loop(0, B_PER_W, reduce_one, 0)

    pltpu.sync_copy(acc_v, out_hbm.at[pl.ds(wid * B_PER_W, B_PER_W)])


@functools.cache
def _gather_sum():
    return pl.kernel(
        _gather_sum_body,
        out_type=jax.ShapeDtypeStruct((BATCH, EMBED_DIM), jnp.float32),
        mesh=plsc.VectorSubcoreMesh(core_axis_name="c", subcore_axis_name="s"),
        compiler_params=pltpu.CompilerParams(use_tc_tiling_on_sc=False),
        scratch_types=[
            pltpu.VMEM((N_CHUNKS, IDX_CHUNK), jnp.int32),
            pltpu.VMEM((IDX_PER_W, EMBED_DIM), jnp.float32),
            pltpu.VMEM((B_PER_W, EMBED_DIM), jnp.float32),
            pltpu.SemaphoreType.DMA,
        ],
    )


def _matmul_body(e_ref, w_ref, b_ref, o_ref):
    o_ref[...] = (
        lax.dot_general(
            e_ref[...],
            w_ref[...],
            dimension_numbers=(((1,), (1,)), ((), ())),
            preferred_element_type=jnp.float32,
        )
        + b_ref[...]
    )


def _project(embeds, linear_w, linear_b2d):
    n_vt = pl.cdiv(VOCAB, VT)
    return pl.pallas_call(
        _matmul_body,
        grid=(n_vt,),
        in_specs=[
            pl.BlockSpec((BATCH, EMBED_DIM), lambda j: (0, 0)),
            pl.BlockSpec((VT, EMBED_DIM), lambda j: (j, 0)),
            pl.BlockSpec((1, VT), lambda j: (0, j)),
        ],
        out_specs=pl.BlockSpec((BATCH, VT), lambda j: (0, j)),
        out_shape=jax.ShapeDtypeStruct((BATCH, VOCAB), jnp.float32),
    )(embeds, linear_w, linear_b2d)


def kernel(inputs, labels, embed_table, linear_w, linear_b):
    del labels
    embeds = jnp.take(embed_table, inputs, axis=0).sum(axis=1)
    return _project(embeds, linear_w, linear_b.reshape(1, VOCAB))


# speedup vs baseline: 2.2327x; 2.1562x over previous
import functools

import jax
import jax.numpy as jnp
from jax import lax
from jax.experimental import pallas as pl
from jax.experimental.pallas import tpu as pltpu
from jax.experimental.pallas import tpu_sc as plsc

VOCAB = 100000
EMBED_DIM = 64
BATCH = 1024
CTX = 20

NUM_CORES = 2
NUM_SUBCORES = 16
NW = NUM_CORES * NUM_SUBCORES
B_PER_W = BATCH // NW
IDX_PER_W = B_PER_W * CTX
IDX_CHUNK = 128
N_CHUNKS = IDX_PER_W // IDX_CHUNK
D_SLICES = EMBED_DIM // 16

VT = 2048


def _gather_sum_body(idx_hbm, table_hbm, out_hbm, idx_v, rows_v, acc_v, sem):
    wid = lax.axis_index("s") * NUM_CORES + lax.axis_index("c")

    pltpu.sync_copy(idx_hbm.at[wid], idx_v)

    copies = []
    for j in range(N_CHUNKS):
        copies.append(
            pltpu.async_copy(
                table_hbm.at[idx_v.at[j]],
                rows_v.at[pl.ds(j * IDX_CHUNK, IDX_CHUNK)],
                sem,
            )
        )
    for c in copies:
        c.wait()

    def reduce_one(b, _):
        base = b * CTX
        for d in range(D_SLICES):
            sl = pl.ds(d * 16, 16)
            acc = rows_v[base, sl]
            for c in range(1, CTX):
                acc = acc + rows_v[base + c, sl]
            acc_v[b, sl] = acc
        return 0

    lax.fori_loop(0, B_PER_W, reduce_one, 0)

    pltpu.sync_copy(acc_v, out_hbm.at[pl.ds(wid * B_PER_W, B_PER_W)])


@functools.cache
def _gather_sum():
    return pl.kernel(
        _gather_sum_body,
        out_type=jax.ShapeDtypeStruct((BATCH, EMBED_DIM), jnp.float32),
        mesh=plsc.VectorSubcoreMesh(core_axis_name="c", subcore_axis_name="s"),
        compiler_params=pltpu.CompilerParams(use_tc_tiling_on_sc=False),
        scratch_types=[
            pltpu.VMEM((N_CHUNKS, IDX_CHUNK), jnp.int32),
            pltpu.VMEM((IDX_PER_W, EMBED_DIM), jnp.float32),
            pltpu.VMEM((B_PER_W, EMBED_DIM), jnp.float32),
            pltpu.SemaphoreType.DMA,
        ],
    )


def _matmul_body(wt_ref, e_ref, b_ref, o_ref):
    o_ref[...] = (
        lax.dot_general(
            wt_ref[...],
            e_ref[...],
            dimension_numbers=(((0,), (1,)), ((), ())),
            preferred_element_type=jnp.float32,
        )
        + b_ref[...]
    )


def _project(embeds, linear_w, linear_b):
    n_vt = pl.cdiv(VOCAB, VT)
    out_t = pl.pallas_call(
        _matmul_body,
        grid=(n_vt,),
        in_specs=[
            pl.BlockSpec((EMBED_DIM, VT), lambda j: (0, j)),
            pl.BlockSpec((BATCH, EMBED_DIM), lambda j: (0, 0)),
            pl.BlockSpec((VT, 1), lambda j: (j, 0)),
        ],
        out_specs=pl.BlockSpec((VT, BATCH), lambda j: (j, 0)),
        out_shape=jax.ShapeDtypeStruct((VOCAB, BATCH), jnp.float32),
    )(linear_w.T, embeds, linear_b.reshape(VOCAB, 1))
    return out_t.T


def kernel(inputs, labels, embed_table, linear_w, linear_b):
    del labels
    idx = inputs.astype(jnp.int32).reshape(NW, N_CHUNKS, IDX_CHUNK)
    embeds = _gather_sum()(idx, embed_table)
    return _project(embeds, linear_w, linear_b)


# speedup vs baseline: 2.8269x; 1.2661x over previous
import functools

import jax
import jax.numpy as jnp
from jax import lax
from jax.experimental import pallas as pl
from jax.experimental.pallas import tpu as pltpu
from jax.experimental.pallas import tpu_sc as plsc

VOCAB = 100000
EMBED_DIM = 64
BATCH = 1024
CTX = 20

NUM_CORES = 2
NUM_SUBCORES = 16
NW = NUM_CORES * NUM_SUBCORES
B_PER_W = BATCH // NW
IDX_PER_W = B_PER_W * CTX
IDX_CHUNK = 128
N_CHUNKS = IDX_PER_W // IDX_CHUNK
D_SLICES = EMBED_DIM // 16

VT = 2048


def _gather_sum_body(idx_hbm, table_hbm, out_hbm, idx_v, rows_v, acc_v, sem):
    wid = lax.axis_index("s") * NUM_CORES + lax.axis_index("c")

    pltpu.sync_copy(idx_hbm.at[wid], idx_v)

    copies = []
    for j in range(N_CHUNKS):
        copies.append(
            pltpu.async_copy(
                table_hbm.at[idx_v.at[j]],
                rows_v.at[pl.ds(j * IDX_CHUNK, IDX_CHUNK)],
                sem,
            )
        )
    for c in copies:
        c.wait()

    def reduce_one(b, _):
        base = b * CTX
        for d in range(D_SLICES):
            sl = pl.ds(d * 16, 16)
            acc = rows_v[base, sl]
            for c in range(1, CTX):
                acc = acc + rows_v[base + c, sl]
            acc_v[b, sl] = acc
        return 0

    lax.fori_loop(0, B_PER_W, reduce_one, 0)

    pltpu.sync_copy(acc_v, out_hbm.at[pl.ds(wid * B_PER_W, B_PER_W)])


@functools.cache
def _gather_sum():
    return pl.kernel(
        _gather_sum_body,
        out_type=jax.ShapeDtypeStruct((BATCH, EMBED_DIM), jnp.float32),
        mesh=plsc.VectorSubcoreMesh(core_axis_name="c", subcore_axis_name="s"),
        compiler_params=pltpu.CompilerParams(use_tc_tiling_on_sc=False),
        scratch_types=[
            pltpu.VMEM((N_CHUNKS, IDX_CHUNK), jnp.int32),
            pltpu.VMEM((IDX_PER_W, 128), jnp.float32),
            pltpu.VMEM((B_PER_W, EMBED_DIM), jnp.float32),
            pltpu.SemaphoreType.DMA,
        ],
    )


def _matmul_body(wt_ref, e_ref, b_ref, o_ref):
    o_ref[...] = (
        lax.dot_general(
            wt_ref[...],
            e_ref[...],
            dimension_numbers=(((0,), (1,)), ((), ())),
            preferred_element_type=jnp.float32,
        )
        + jnp.transpose(b_ref[...])
    )


def _project(embeds, linear_w, linear_b):
    n_vt = pl.cdiv(VOCAB, VT)
    out_t = pl.pallas_call(
        _matmul_body,
        grid=(n_vt,),
        in_specs=[
            pl.BlockSpec((EMBED_DIM, VT), lambda j: (0, j)),
            pl.BlockSpec((BATCH, EMBED_DIM), lambda j: (0, 0)),
            pl.BlockSpec((1, VT), lambda j: (0, j)),
        ],
        out_specs=pl.BlockSpec((VT, BATCH), lambda j: (j, 0)),
        out_shape=jax.ShapeDtypeStruct((VOCAB, BATCH), jnp.float32),
    )(linear_w.T, embeds, linear_b.reshape(1, VOCAB))
    return out_t.T


def kernel(inputs, labels, embed_table, linear_w, linear_b):
    del labels
    idx = inputs.astype(jnp.int32).reshape(NW, N_CHUNKS, IDX_CHUNK)
    table128 = jnp.pad(embed_table, ((0, 0), (0, 128 - EMBED_DIM)))
    embeds = _gather_sum()(idx, table128)
    return _project(embeds, linear_w, linear_b)


# speedup vs baseline: 3.0744x; 1.0876x over previous
import functools

import jax
import jax.numpy as jnp
from jax import lax
from jax.experimental import pallas as pl
from jax.experimental.pallas import tpu as pltpu
from jax.experimental.pallas import tpu_sc as plsc

VOCAB = 100000
EMBED_DIM = 64
BATCH = 1024
CTX = 20

NUM_CORES = 2
NUM_SUBCORES = 16
NW = NUM_CORES * NUM_SUBCORES
B_PER_W = BATCH // NW
IDX_PER_W = B_PER_W * CTX
IDX_CHUNK = 128
N_CHUNKS = IDX_PER_W // IDX_CHUNK
D_SLICES = EMBED_DIM // 16

VT = 2048


def _gather_sum_body(idx_hbm, table_hbm, out_hbm, idx_v, rows_v, acc_v, sem):
    wid = lax.axis_index("s") * NUM_CORES + lax.axis_index("c")

    pltpu.sync_copy(idx_hbm.at[wid], idx_v)

    copies = []
    for j in range(N_CHUNKS):
        copies.append(
            pltpu.async_copy(
                table_hbm.at[idx_v.at[j]],
                rows_v.at[pl.ds(j * IDX_CHUNK, IDX_CHUNK)],
                sem,
            )
        )
    for c in copies:
        c.wait()

    def reduce_one(b, _):
        base = b * CTX
        for d in range(D_SLICES):
            sl = pl.ds(d * 16, 16)
            acc = rows_v[base, sl]
            for c in range(1, CTX):
                acc = acc + rows_v[base + c, sl]
            acc_v[b, sl] = acc
        return 0

    lax.fori_loop(0, B_PER_W, reduce_one, 0)

    pltpu.sync_copy(acc_v, out_hbm.at[pl.ds(wid * B_PER_W, B_PER_W)])


@functools.cache
def _gather_sum():
    return pl.kernel(
        _gather_sum_body,
        out_type=jax.ShapeDtypeStruct((BATCH, EMBED_DIM), jnp.float32),
        mesh=plsc.VectorSubcoreMesh(core_axis_name="c", subcore_axis_name="s"),
        compiler_params=pltpu.CompilerParams(use_tc_tiling_on_sc=False),
        scratch_types=[
            pltpu.VMEM((N_CHUNKS, IDX_CHUNK), jnp.int32),
            pltpu.VMEM((IDX_PER_W, 128), jnp.float32),
            pltpu.VMEM((B_PER_W, EMBED_DIM), jnp.float32),
            pltpu.SemaphoreType.DMA,
        ],
    )


VT2 = 4096


def _padtable_body(tt_ref, o_ref):
    t = jnp.transpose(tt_ref[...])
    o_ref[...] = jnp.concatenate(
        [t, jnp.zeros((t.shape[0], 128 - EMBED_DIM), jnp.float32)], axis=1
    )


def _pad_table(embed_table):
    n = pl.cdiv(VOCAB, VT2)
    return pl.pallas_call(
        _padtable_body,
        grid=(n,),
        in_specs=[pl.BlockSpec((EMBED_DIM, VT2), lambda j: (0, j))],
        out_specs=pl.BlockSpec((VT2, 128), lambda j: (j, 0)),
        out_shape=jax.ShapeDtypeStruct((VOCAB, 128), jnp.float32),
    )(embed_table.T)


def _matmul_body(wt_ref, e_ref, b_ref, o_ref):
    o_ref[...] = (
        lax.dot_general(
            wt_ref[...],
            e_ref[...],
            dimension_numbers=(((0,), (1,)), ((), ())),
            preferred_element_type=jnp.float32,
        )
        + jnp.transpose(b_ref[...])
    )


def _project(embeds, linear_w, linear_b):
    n_vt = pl.cdiv(VOCAB, VT)
    out_t = pl.pallas_call(
        _matmul_body,
        grid=(n_vt,),
        in_specs=[
            pl.BlockSpec((EMBED_DIM, VT), lambda j: (0, j)),
            pl.BlockSpec((BATCH, EMBED_DIM), lambda j: (0, 0)),
            pl.BlockSpec((1, VT), lambda j: (0, j)),
        ],
        out_specs=pl.BlockSpec((VT, BATCH), lambda j: (j, 0)),
        out_shape=jax.ShapeDtypeStruct((VOCAB, BATCH), jnp.float32),
    )(linear_w.T, embeds, linear_b.reshape(1, VOCAB))
    return out_t.T


def kernel(inputs, labels, embed_table, linear_w, linear_b):
    del labels
    idx = inputs.astype(jnp.int32).reshape(NW, N_CHUNKS, IDX_CHUNK)
    table128 = _pad_table(embed_table)
    embeds = _gather_sum()(idx, table128)
    return _project(embeds, linear_w, linear_b)


# speedup vs baseline: 3.0941x; 1.0064x over previous
import functools

import jax
import jax.numpy as jnp
from jax import lax
from jax.experimental import pallas as pl
from jax.experimental.pallas import tpu as pltpu
from jax.experimental.pallas import tpu_sc as plsc

VOCAB = 100000
EMBED_DIM = 64
BATCH = 1024
CTX = 20

NUM_CORES = 2
NUM_SUBCORES = 16
NW = NUM_CORES * NUM_SUBCORES
B_PER_W = BATCH // NW
IDX_PER_W = B_PER_W * CTX
IDX_CHUNK = 128
N_CHUNKS = IDX_PER_W // IDX_CHUNK
D_SLICES = EMBED_DIM // 16

VT = 4096


def _gather_sum_body(idx_hbm, table_hbm, out_hbm, idx_v, rows_v, acc_v, sem):
    wid = lax.axis_index("s") * NUM_CORES + lax.axis_index("c")

    pltpu.sync_copy(idx_hbm.at[wid], idx_v)

    copies = []
    for j in range(N_CHUNKS):
        copies.append(
            pltpu.async_copy(
                table_hbm.at[idx_v.at[j]],
                rows_v.at[pl.ds(j * IDX_CHUNK, IDX_CHUNK)],
                sem,
            )
        )
    for c in copies:
        c.wait()

    def reduce_one(b, _):
        base = b * CTX
        for d in range(D_SLICES):
            sl = pl.ds(d * 16, 16)
            acc = rows_v[base, sl]
            for c in range(1, CTX):
                acc = acc + rows_v[base + c, sl]
            acc_v[b, sl] = acc
        return 0

    lax.fori_loop(0, B_PER_W, reduce_one, 0)

    pltpu.sync_copy(acc_v, out_hbm.at[pl.ds(wid * B_PER_W, B_PER_W)])


@functools.cache
def _gather_sum():
    return pl.kernel(
        _gather_sum_body,
        out_type=jax.ShapeDtypeStruct((BATCH, EMBED_DIM), jnp.float32),
        mesh=plsc.VectorSubcoreMesh(core_axis_name="c", subcore_axis_name="s"),
        compiler_params=pltpu.CompilerParams(use_tc_tiling_on_sc=False),
        scratch_types=[
            pltpu.VMEM((N_CHUNKS, IDX_CHUNK), jnp.int32),
            pltpu.VMEM((IDX_PER_W, 128), jnp.float32),
            pltpu.VMEM((B_PER_W, EMBED_DIM), jnp.float32),
            pltpu.SemaphoreType.DMA,
        ],
    )


VT2 = 4096


def _padtable_body(tt_ref, o_ref):
    t = jnp.transpose(tt_ref[...])
    o_ref[...] = jnp.concatenate(
        [t, jnp.zeros((t.shape[0], 128 - EMBED_DIM), jnp.float32)], axis=1
    )


def _pad_table(embed_table):
    n = pl.cdiv(VOCAB, VT2)
    return pl.pallas_call(
        _padtable_body,
        grid=(n,),
        in_specs=[pl.BlockSpec((EMBED_DIM, VT2), lambda j: (0, j))],
        out_specs=pl.BlockSpec((VT2, 128), lambda j: (j, 0)),
        out_shape=jax.ShapeDtypeStruct((VOCAB, 128), jnp.float32),
    )(embed_table.T)


def _matmul_body(wt_ref, e_ref, b_ref, o_ref):
    o_ref[...] = (
        lax.dot_general(
            wt_ref[...],
            e_ref[...],
            dimension_numbers=(((0,), (1,)), ((), ())),
            preferred_element_type=jnp.float32,
        )
        + jnp.transpose(b_ref[...])
    )


def _project(embeds, linear_w, linear_b):
    n_vt = pl.cdiv(VOCAB, VT)
    out_t = pl.pallas_call(
        _matmul_body,
        grid=(n_vt,),
        in_specs=[
            pl.BlockSpec((EMBED_DIM, VT), lambda j: (0, j)),
            pl.BlockSpec((BATCH, EMBED_DIM), lambda j: (0, 0)),
            pl.BlockSpec((1, VT), lambda j: (0, j)),
        ],
        out_specs=pl.BlockSpec((VT, BATCH), lambda j: (j, 0)),
        out_shape=jax.ShapeDtypeStruct((VOCAB, BATCH), jnp.float32),
    )(linear_w.T, embeds, linear_b.reshape(1, VOCAB))
    return out_t.T


def kernel(inputs, labels, embed_table, linear_w, linear_b):
    del labels
    idx = inputs.astype(jnp.int32).reshape(NW, N_CHUNKS, IDX_CHUNK)
    table128 = _pad_table(embed_table)
    embeds = _gather_sum()(idx, table128)
    return _project(embeds, linear_w, linear_b)


# speedup vs baseline: 3.2310x; 1.0442x over previous
import functools

import jax
import jax.numpy as jnp
from jax import lax
from jax.experimental import pallas as pl
from jax.experimental.pallas import tpu as pltpu
from jax.experimental.pallas import tpu_sc as plsc

VOCAB = 100000
EMBED_DIM = 64
BATCH = 1024
CTX = 20

NUM_CORES = 2
NUM_SUBCORES = 16
NW = NUM_CORES * NUM_SUBCORES
B_PER_W = BATCH // NW
IDX_PER_W = B_PER_W * CTX
IDX_CHUNK = 128
N_CHUNKS = IDX_PER_W // IDX_CHUNK
D_SLICES = EMBED_DIM // 16

VT = 5120


def _gather_sum_body(idx_hbm, table_hbm, out_hbm, idx_v, rows_v, acc_v, sem):
    wid = lax.axis_index("s") * NUM_CORES + lax.axis_index("c")

    pltpu.sync_copy(idx_hbm.at[wid], idx_v)

    copies = []
    for j in range(N_CHUNKS):
        copies.append(
            pltpu.async_copy(
                table_hbm.at[idx_v.at[j]],
                rows_v.at[pl.ds(j * IDX_CHUNK, IDX_CHUNK)],
                sem,
            )
        )
    for c in copies:
        c.wait()

    def reduce_one(b, _):
        base = b * CTX
        for d in range(D_SLICES):
            sl = pl.ds(d * 16, 16)
            acc = rows_v[base, sl]
            for c in range(1, CTX):
                acc = acc + rows_v[base + c, sl]
            acc_v[b, sl] = acc
        return 0

    lax.fori_loop(0, B_PER_W, reduce_one, 0)

    pltpu.sync_copy(acc_v, out_hbm.at[pl.ds(wid * B_PER_W, B_PER_W)])


@functools.cache
def _gather_sum():
    return pl.kernel(
        _gather_sum_body,
        out_type=jax.ShapeDtypeStruct((BATCH, EMBED_DIM), jnp.float32),
        mesh=plsc.VectorSubcoreMesh(core_axis_name="c", subcore_axis_name="s"),
        compiler_params=pltpu.CompilerParams(use_tc_tiling_on_sc=False),
        scratch_types=[
            pltpu.VMEM((N_CHUNKS, IDX_CHUNK), jnp.int32),
            pltpu.VMEM((IDX_PER_W, 128), jnp.float32),
            pltpu.VMEM((B_PER_W, EMBED_DIM), jnp.float32),
            pltpu.SemaphoreType.DMA,
        ],
    )


VT2 = 8192


def _padtable_body(tt_ref, o_ref):
    t = jnp.transpose(tt_ref[...])
    o_ref[...] = jnp.concatenate(
        [t, jnp.zeros((t.shape[0], 128 - EMBED_DIM), jnp.float32)], axis=1
    )


def _pad_table(embed_table):
    n = pl.cdiv(VOCAB, VT2)
    return pl.pallas_call(
        _padtable_body,
        grid=(n,),
        in_specs=[pl.BlockSpec((EMBED_DIM, VT2), lambda j: (0, j))],
        out_specs=pl.BlockSpec((VT2, 128), lambda j: (j, 0)),
        out_shape=jax.ShapeDtypeStruct((VOCAB, 128), jnp.float32),
    )(embed_table.T)


def _matmul_body(wt_ref, e_ref, b_ref, o_ref):
    o_ref[...] = (
        lax.dot_general(
            wt_ref[...],
            e_ref[...],
            dimension_numbers=(((0,), (1,)), ((), ())),
            preferred_element_type=jnp.float32,
        )
        + jnp.transpose(b_ref[...])
    )


def _project(embeds, linear_w, linear_b):
    n_vt = pl.cdiv(VOCAB, VT)
    out_t = pl.pallas_call(
        _matmul_body,
        grid=(n_vt,),
        in_specs=[
            pl.BlockSpec((EMBED_DIM, VT), lambda j: (0, j)),
            pl.BlockSpec((BATCH, EMBED_DIM), lambda j: (0, 0)),
            pl.BlockSpec((1, VT), lambda j: (0, j)),
        ],
        out_specs=pl.BlockSpec((VT, BATCH), lambda j: (j, 0)),
        out_shape=jax.ShapeDtypeStruct((VOCAB, BATCH), jnp.float32),
    )(linear_w.T, embeds, linear_b.reshape(1, VOCAB))
    return out_t.T


def kernel(inputs, labels, embed_table, linear_w, linear_b):
    del labels
    idx = inputs.astype(jnp.int32).reshape(NW, N_CHUNKS, IDX_CHUNK)
    table128 = _pad_table(embed_table)
    embeds = _gather_sum()(idx, table128)
    return _project(embeds, linear_w, linear_b)


# speedup vs baseline: 3.2584x; 1.0085x over previous
import functools

import jax
import jax.numpy as jnp
from jax import lax
from jax.experimental import pallas as pl
from jax.experimental.pallas import tpu as pltpu
from jax.experimental.pallas import tpu_sc as plsc

VOCAB = 100000
EMBED_DIM = 64
BATCH = 1024
CTX = 20

NUM_CORES = 2
NUM_SUBCORES = 16
NW = NUM_CORES * NUM_SUBCORES
B_PER_W = BATCH // NW
IDX_PER_W = B_PER_W * CTX
IDX_CHUNK = 128
N_CHUNKS = IDX_PER_W // IDX_CHUNK
D_SLICES = EMBED_DIM // 16

VT = 5120


def _gather_sum_body(idx_hbm, table_hbm, out_hbm, idx_v, rows_v, acc_v, sem):
    wid = lax.axis_index("s") * NUM_CORES + lax.axis_index("c")

    pltpu.sync_copy(idx_hbm.at[wid], idx_v)

    copies = []
    for j in range(N_CHUNKS):
        copies.append(
            pltpu.async_copy(
                table_hbm.at[idx_v.at[j]],
                rows_v.at[pl.ds(j * IDX_CHUNK, IDX_CHUNK)],
                sem,
            )
        )
    for c in copies:
        c.wait()

    def reduce_one(b, _):
        base = b * CTX
        for d in range(D_SLICES):
            sl = pl.ds(d * 16, 16)
            acc = rows_v[base, sl]
            for c in range(1, CTX):
                acc = acc + rows_v[base + c, sl]
            acc_v[b, sl] = acc
        return 0

    lax.fori_loop(0, B_PER_W, reduce_one, 0)

    pltpu.sync_copy(acc_v, out_hbm.at[pl.ds(wid * B_PER_W, B_PER_W)])


@functools.cache
def _gather_sum():
    return pl.kernel(
        _gather_sum_body,
        out_type=jax.ShapeDtypeStruct((BATCH, EMBED_DIM), jnp.float32),
        mesh=plsc.VectorSubcoreMesh(core_axis_name="c", subcore_axis_name="s"),
        compiler_params=pltpu.CompilerParams(use_tc_tiling_on_sc=False),
        scratch_types=[
            pltpu.VMEM((N_CHUNKS, IDX_CHUNK), jnp.int32),
            pltpu.VMEM((IDX_PER_W, 128), jnp.float32),
            pltpu.VMEM((B_PER_W, EMBED_DIM), jnp.float32),
            pltpu.SemaphoreType.DMA,
        ],
    )


VT2 = 16384


def _padtable_body(tt_ref, o_ref):
    t = jnp.transpose(tt_ref[...])
    o_ref[...] = jnp.concatenate(
        [t, jnp.zeros((t.shape[0], 128 - EMBED_DIM), jnp.float32)], axis=1
    )


def _pad_table(embed_table):
    n = pl.cdiv(VOCAB, VT2)
    return pl.pallas_call(
        _padtable_body,
        grid=(n,),
        in_specs=[pl.BlockSpec((EMBED_DIM, VT2), lambda j: (0, j))],
        out_specs=pl.BlockSpec((VT2, 128), lambda j: (j, 0)),
        out_shape=jax.ShapeDtypeStruct((VOCAB, 128), jnp.float32),
    )(embed_table.T)


def _matmul_body(wt_ref, e_ref, b_ref, o_ref):
    o_ref[...] = (
        lax.dot_general(
            wt_ref[...],
            e_ref[...],
            dimension_numbers=(((0,), (1,)), ((), ())),
            preferred_element_type=jnp.float32,
        )
        + jnp.transpose(b_ref[...])
    )


def _project(embeds, linear_w, linear_b):
    n_vt = pl.cdiv(VOCAB, VT)
    out_t = pl.pallas_call(
        _matmul_body,
        grid=(n_vt,),
        in_specs=[
            pl.BlockSpec((EMBED_DIM, VT), lambda j: (0, j)),
            pl.BlockSpec((BATCH, EMBED_DIM), lambda j: (0, 0)),
            pl.BlockSpec((1, VT), lambda j: (0, j)),
        ],
        out_specs=pl.BlockSpec((VT, BATCH), lambda j: (j, 0)),
        out_shape=jax.ShapeDtypeStruct((VOCAB, BATCH), jnp.float32),
    )(linear_w.T, embeds, linear_b.reshape(1, VOCAB))
    return out_t.T


def kernel(inputs, labels, embed_table, linear_w, linear_b):
    del labels
    idx = inputs.astype(jnp.int32).reshape(NW, N_CHUNKS, IDX_CHUNK)
    table128 = _pad_table(embed_table)
    embeds = _gather_sum()(idx, table128)
    return _project(embeds, linear_w, linear_b)
